# baseline (speedup 1.0000x reference)
import jax
import jax.numpy as jnp
from jax import lax
from jax.experimental import pallas as pl
from jax.experimental.pallas import tpu as pltpu
from jax.experimental.pallas import tpu_sc as plsc

N_DEPTH = 7
EMBED_DIM = 16
N_POINTS = 16384
OUT_DIM = N_DEPTH * EMBED_DIM
LANES = 16

NUM_CORES = 2
NUM_SUBCORES = 16
NUM_WORKERS = NUM_CORES * NUM_SUBCORES
PTS_PER_WORKER = N_POINTS // NUM_WORKERS
N_PASS = 4
HP = PTS_PER_WORKER // N_PASS
GROUPS = HP // LANES
CT_PER_PASS = HP // 128
ROW_GROUPS = OUT_DIM // 8
N_COL_TILES = N_POINTS // 128

STREAM_DEPTHS = [5, 6, 7]
LOCAL_DEPTHS = [1, 2, 3, 4]
N_STREAM = len(STREAM_DEPTHS)

_ONE_MINUS_EPS = 1.0 - 1e-06


def _flat_tile_view(wt):
    v = wt.shape[1]
    return (wt.reshape(2, 8, v // 128, 128)
              .transpose(0, 2, 1, 3)
              .reshape(-1))


def _body(inp_t, w1, wt2, wt3, wt4, f5, f6, f7, out_flat,
          coords_v, t1_v, t2_v, t3_v, t4_v,
          p5a, p6a, p7a, ova, p5b, p6b, p7b, ovb,
          gsem_a, gsem_b, wsem_a, wsem_b):
    flats = {5: f5, 6: f6, 7: f7}
    bufs = [({5: p5a, 6: p6a, 7: p7a}, ova, gsem_a, wsem_a),
            ({5: p5b, 6: p6b, 7: p7b}, ovb, gsem_b, wsem_b)]
    local_tv = {1: t1_v, 2: t2_v, 3: t3_v, 4: t4_v}

    wid = lax.axis_index("s") * NUM_CORES + lax.axis_index("c")
    base = wid * PTS_PER_WORKER

    stage = [
        pltpu.async_copy(inp_t.at[:, pl.ds(base, PTS_PER_WORKER)],
                         coords_v, wsem_a),
        pltpu.async_copy(w1, t1_v, wsem_a),
        pltpu.async_copy(wt2, t2_v, wsem_a),
        pltpu.async_copy(wt3, t3_v, wsem_a),
        pltpu.async_copy(wt4, t4_v, wsem_a),
    ]
    for cp in stage:
        cp.wait()

    esplat = [jnp.full((LANES,), e, jnp.int32) for e in range(EMBED_DIM)]

    def compute(ps, pidx, out_v):
        @pl.loop(0, GROUPS)
        def _grp(i):
            p0 = ps * HP + i * LANES
            pm = i * LANES
            cell = []
            for a in range(3):
                v = coords_v[a, pl.ds(p0, LANES)]
                s = (v + 1.0) * 0.5
                s = jnp.minimum(jnp.maximum(s, 0.0), _ONE_MINUS_EPS)
                cell.append((s * 128.0).astype(jnp.int32))
            x7, y7, z7 = cell
            for d in range(1, N_DEPTH + 1):
                sh = N_DEPTH - d
                idx = (((z7 >> sh) << (2 * d))
                       + ((y7 >> sh) << d)
                       + (x7 >> sh))
                if d in LOCAL_DEPTHS:
                    tv = local_tv[d]
                    for e in range(EMBED_DIM):
                        if d == 1:
                            row_v = plsc.load_gather(tv, [idx, esplat[e]])
                        else:
                            row_v = plsc.load_gather(tv, [esplat[e], idx])
                        row = (d - 1) * EMBED_DIM + e
                        off = ((row // 8) * 1024 + (row % 8) * 128 + pm)
                        out_v[pl.ds(off, LANES)] = row_v
                else:
                    vv = (2 ** d) ** 3
                    rpart = ((idx >> 7) << 10) + (idx & 127)
                    for e in range(EMBED_DIM):
                        a_c = (e // 8) * (vv * 8) + (e % 8) * 128
                        off = (e // 8) * 1024 + (e % 8) * 128 + pm
                        pidx[d][pl.ds(off, LANES)] = rpart + a_c

    def fire(pidx, out_v, sem):
        copies = []
        for d in STREAM_DEPTHS:
            for h in range(2):
                idx_ref = pidx[d].at[pl.ds(h * 1024, 1024)]
                rg = 2 * (d - 1) + h
                dst = out_v.at[pl.ds(rg * 1024, 1024)]
                copies.append(
                    pltpu.async_copy(flats[d].at[idx_ref], dst, sem))
        return copies

    def writeback(ps, out_v, wsem):
        gct = wid * N_PASS + ps
        return [pltpu.async_copy(
                    out_v.at[pl.ds(rg * 1024, 1024)],
                    out_flat.at[pl.ds((rg * N_COL_TILES + gct) * 1024,
                                      1024)],
                    wsem)
                for rg in range(ROW_GROUPS)]

    prev = None
    wpending = [None, None]
    for ps in range(N_PASS):
        pidx, out_v, sem, wsem = bufs[ps % 2]
        if wpending[ps % 2] is not None:
            for cp in wpending[ps % 2]:
                cp.wait()
            wpending[ps % 2] = None
        compute(ps, pidx, out_v)
        if prev is not None:
            pps, pcopies, pov, pwsem, ppar = prev
            for cp in pcopies:
                cp.wait()
            wpending[ppar] = writeback(pps, pov, pwsem)
        prev = (ps, fire(pidx, out_v, sem), out_v, wsem, ps % 2)
    pps, pcopies, pov, pwsem, ppar = prev
    for cp in pcopies:
        cp.wait()
    for cp in writeback(pps, pov, pwsem):
        cp.wait()
    for w in wpending:
        if w is not None:
            for cp in w:
                cp.wait()


@jax.jit
def kernel(input, W1, W2, W3, W4, W5, W6, W7):
    tables = [W1, W2, W3, W4, W5, W6, W7]
    wts = [w.T for w in tables]
    flats = [_flat_tile_view(wts[d - 1]) for d in STREAM_DEPTHS]

    mesh = plsc.VectorSubcoreMesh(
        core_axis_name="c", subcore_axis_name="s",
        num_cores=NUM_CORES, num_subcores=NUM_SUBCORES)
    f = pl.kernel(
        _body,
        out_type=jax.ShapeDtypeStruct((OUT_DIM * N_POINTS,), jnp.float32),
        mesh=mesh,
        scratch_types=[
            pltpu.VMEM((3, PTS_PER_WORKER), jnp.float32),
            pltpu.VMEM((8, EMBED_DIM), jnp.float32),
            pltpu.VMEM((EMBED_DIM, 64), jnp.float32),
            pltpu.VMEM((EMBED_DIM, 512), jnp.float32),
            pltpu.VMEM((EMBED_DIM, 4096), jnp.float32),
        ] + [pltpu.VMEM((EMBED_DIM * HP,), jnp.int32)
             for _ in range(N_STREAM)] + [
            pltpu.VMEM((ROW_GROUPS * 1024,), jnp.float32),
        ] + [pltpu.VMEM((EMBED_DIM * HP,), jnp.int32)
             for _ in range(N_STREAM)] + [
            pltpu.VMEM((ROW_GROUPS * 1024,), jnp.float32),
            pltpu.SemaphoreType.DMA,
            pltpu.SemaphoreType.DMA,
            pltpu.SemaphoreType.DMA,
            pltpu.SemaphoreType.DMA,
        ],
        compiler_params=pltpu.CompilerParams(needs_layout_passes=False),
    )
    out_flat = f(input.T, W1, wts[1], wts[2], wts[3], *flats)
    return (out_flat.reshape(ROW_GROUPS, N_COL_TILES, 8, 128)
            .transpose(0, 2, 1, 3)
            .reshape(OUT_DIM, N_POINTS)
            .T)


# speedup vs baseline: 1.0204x; 1.0204x over previous
import jax
import jax.numpy as jnp
from jax import lax
from jax.experimental import pallas as pl
from jax.experimental.pallas import tpu as pltpu
from jax.experimental.pallas import tpu_sc as plsc

N_DEPTH = 7
EMBED_DIM = 16
N_POINTS = 16384
OUT_DIM = N_DEPTH * EMBED_DIM
LANES = 16

NUM_CORES = 2
NUM_SUBCORES = 16
NUM_WORKERS = NUM_CORES * NUM_SUBCORES
PTS_PER_WORKER = N_POINTS // NUM_WORKERS
N_PASS = 4
HP = PTS_PER_WORKER // N_PASS
GROUPS = HP // LANES
CT_PER_PASS = HP // 128
ROW_GROUPS = OUT_DIM // 8
N_COL_TILES = N_POINTS // 128

STREAM_DEPTHS = [5, 6, 7]
LOCAL_DEPTHS = [1, 2, 3, 4]
N_STREAM = len(STREAM_DEPTHS)

_ONE_MINUS_EPS = 1.0 - 1e-06


def _flat_tile_view(wt):
    v = wt.shape[1]
    return (wt.reshape(2, 8, v // 128, 128)
              .transpose(0, 2, 1, 3)
              .reshape(-1))


def _body(inp_t, w1, wt2, wt3, wt4, f5, f6, f7, out_flat,
          coords_v, t1_v, t2_v, t3_v, t4_v,
          p5a, p6a, p7a, ova, p5b, p6b, p7b, ovb,
          gsem_a, gsem_b, wsem_a, wsem_b):
    flats = {5: f5, 6: f6, 7: f7}
    bufs = [({5: p5a, 6: p6a, 7: p7a}, ova, gsem_a, wsem_a),
            ({5: p5b, 6: p6b, 7: p7b}, ovb, gsem_b, wsem_b)]
    local_tv = {1: t1_v, 2: t2_v, 3: t3_v, 4: t4_v}

    wid = lax.axis_index("s") * NUM_CORES + lax.axis_index("c")
    base = wid * PTS_PER_WORKER

    pltpu.sync_copy(inp_t.at[:, pl.ds(base, PTS_PER_WORKER)], coords_v)
    pltpu.sync_copy(w1, t1_v)
    pltpu.sync_copy(wt2, t2_v)
    pltpu.sync_copy(wt3, t3_v)
    pltpu.sync_copy(wt4, t4_v)

    esplat = [jnp.full((LANES,), e, jnp.int32) for e in range(EMBED_DIM)]

    def compute(ps, pidx, out_v):
        @pl.loop(0, GROUPS)
        def _grp(i):
            p0 = ps * HP + i * LANES
            pm = i * LANES
            cell = []
            for a in range(3):
                v = coords_v[a, pl.ds(p0, LANES)]
                s = (v + 1.0) * 0.5
                s = jnp.minimum(jnp.maximum(s, 0.0), _ONE_MINUS_EPS)
                cell.append((s * 128.0).astype(jnp.int32))
            x7, y7, z7 = cell
            for d in range(1, N_DEPTH + 1):
                sh = N_DEPTH - d
                idx = (((z7 >> sh) << (2 * d))
                       + ((y7 >> sh) << d)
                       + (x7 >> sh))
                if d in LOCAL_DEPTHS:
                    tv = local_tv[d]
                    for e in range(EMBED_DIM):
                        if d == 1:
                            row_v = plsc.load_gather(tv, [idx, esplat[e]])
                        else:
                            row_v = plsc.load_gather(tv, [esplat[e], idx])
                        row = (d - 1) * EMBED_DIM + e
                        off = ((row // 8) * 1024 + (row % 8) * 128 + pm)
                        out_v[pl.ds(off, LANES)] = row_v
                else:
                    vv = (2 ** d) ** 3
                    rpart = ((idx >> 7) << 10) + (idx & 127)
                    for e in range(EMBED_DIM):
                        a_c = (e // 8) * (vv * 8) + (e % 8) * 128
                        off = (e // 8) * 1024 + (e % 8) * 128 + pm
                        pidx[d][pl.ds(off, LANES)] = rpart + a_c

    def fire(pidx, out_v, sem):
        copies = []
        for d in STREAM_DEPTHS:
            for h in range(2):
                idx_ref = pidx[d].at[pl.ds(h * 1024, 1024)]
                rg = 2 * (d - 1) + h
                dst = out_v.at[pl.ds(rg * 1024, 1024)]
                copies.append(
                    pltpu.async_copy(flats[d].at[idx_ref], dst, sem))
        return copies

    def writeback(ps, out_v, wsem):
        gct = wid * N_PASS + ps
        return [pltpu.async_copy(
                    out_v.at[pl.ds(rg * 1024, 1024)],
                    out_flat.at[pl.ds((rg * N_COL_TILES + gct) * 1024,
                                      1024)],
                    wsem)
                for rg in range(ROW_GROUPS)]

    prev = None
    wpending = [None, None]
    for ps in range(N_PASS):
        pidx, out_v, sem, wsem = bufs[ps % 2]
        if wpending[ps % 2] is not None:
            for cp in wpending[ps % 2]:
                cp.wait()
            wpending[ps % 2] = None
        compute(ps, pidx, out_v)
        if prev is not None:
            pps, pcopies, pov, pwsem, ppar = prev
            for cp in pcopies:
                cp.wait()
            wpending[ppar] = writeback(pps, pov, pwsem)
        prev = (ps, fire(pidx, out_v, sem), out_v, wsem, ps % 2)
    pps, pcopies, pov, pwsem, ppar = prev
    for cp in pcopies:
        cp.wait()
    for cp in writeback(pps, pov, pwsem):
        cp.wait()
    for w in wpending:
        if w is not None:
            for cp in w:
                cp.wait()


@jax.jit
def kernel(input, W1, W2, W3, W4, W5, W6, W7):
    tables = [W1, W2, W3, W4, W5, W6, W7]
    wts = [w.T for w in tables]
    flats = [_flat_tile_view(wts[d - 1]) for d in STREAM_DEPTHS]

    mesh = plsc.VectorSubcoreMesh(
        core_axis_name="c", subcore_axis_name="s",
        num_cores=NUM_CORES, num_subcores=NUM_SUBCORES)
    f = pl.kernel(
        _body,
        out_type=jax.ShapeDtypeStruct((OUT_DIM * N_POINTS,), jnp.float32),
        mesh=mesh,
        scratch_types=[
            pltpu.VMEM((3, PTS_PER_WORKER), jnp.float32),
            pltpu.VMEM((8, EMBED_DIM), jnp.float32),
            pltpu.VMEM((EMBED_DIM, 64), jnp.float32),
            pltpu.VMEM((EMBED_DIM, 512), jnp.float32),
            pltpu.VMEM((EMBED_DIM, 4096), jnp.float32),
        ] + [pltpu.VMEM((EMBED_DIM * HP,), jnp.int32)
             for _ in range(N_STREAM)] + [
            pltpu.VMEM((ROW_GROUPS * 1024,), jnp.float32),
        ] + [pltpu.VMEM((EMBED_DIM * HP,), jnp.int32)
             for _ in range(N_STREAM)] + [
            pltpu.VMEM((ROW_GROUPS * 1024,), jnp.float32),
            pltpu.SemaphoreType.DMA,
            pltpu.SemaphoreType.DMA,
            pltpu.SemaphoreType.DMA,
            pltpu.SemaphoreType.DMA,
        ],
        compiler_params=pltpu.CompilerParams(needs_layout_passes=False),
    )
    out_flat = f(input.T, W1, wts[1], wts[2], wts[3], *flats)
    return (out_flat.reshape(ROW_GROUPS, N_COL_TILES, 8, 128)
            .transpose(0, 2, 1, 3)
            .reshape(OUT_DIM, N_POINTS)
            .T)
